# Initial kernel scaffold; baseline (speedup 1.0000x reference)
import jax
import jax.numpy as jnp
from jax.experimental import pallas as pl


def kernel(x, Wpatch, bpatch, cls_tok, pos_emb, ln1_s, ln1_b, ln2_s, ln2_b, Wq, bq, Wk, bk, Wv, bv, Wo, bo, gate_w, W1, b1, W2, b2, lnf_s, lnf_b, Wc, bc):
    raise NotImplementedError("write your pallas kernel here")


# speedup vs baseline: 1.0743x; 1.0743x over previous
import functools
import jax
import jax.numpy as jnp
from jax.experimental import pallas as pl

B = 8; IMG = 224; P = 16; HID = 768; HEADS = 12; DKV = 64; DFF = 1024
E = 16; DEPTH = 4; NCLS = 1000
S = (IMG // P) ** 2 + 1
T = B * S
TP = 1792
TBLK = 448
CAP = int(2.0 * T / E)
CAPP = 200
NSLOT = E * CAPP
SBLK = 800

_INTERPRET = False


def _pc(body, grid, in_specs, out_specs, out_shape):
    kwargs = {}
    if grid is not None:
        kwargs["grid"] = grid
    if in_specs is not None:
        kwargs["in_specs"] = in_specs
    if out_specs is not None:
        kwargs["out_specs"] = out_specs
    return pl.pallas_call(body, out_shape=out_shape, interpret=_INTERPRET,
                          **kwargs)


def _bf(x):
    return x.astype(jnp.bfloat16).astype(jnp.float32)


def _dot(a, w):
    return jnp.dot(a, w, preferred_element_type=jnp.float32)


def _ln(x, s, b):
    m = x.mean(-1, keepdims=True)
    v = ((x - m) ** 2).mean(-1, keepdims=True)
    return (x - m) / jnp.sqrt(v + 1e-6) * s + b


def _embed_body(p_ref, w_ref, b_ref, o_ref):
    o_ref[...] = _dot(_bf(p_ref[...]), w_ref[...]) + b_ref[...]


def _qkv_body(h_ref, s1_ref, b1_ref, wq_ref, bq_ref, wk_ref, bk_ref,
              wv_ref, bv_ref, q_ref, k_ref, v_ref):
    hn = _ln(h_ref[...], s1_ref[...], b1_ref[...])
    q_ref[...] = _dot(hn, wq_ref[...]) + bq_ref[...]
    k_ref[...] = _dot(hn, wk_ref[...]) + bk_ref[...]
    v_ref[...] = _dot(hn, wv_ref[...]) + bv_ref[...]


def _mix_body(q_ref, k_ref, v_ref, o_ref):
    qb = _bf(q_ref[0])
    kb = _bf(k_ref[0])
    vb = _bf(v_ref[0])
    outs = []
    for hd in range(HEADS):
        qh = qb[:, hd * DKV:(hd + 1) * DKV]
        kh = kb[:, hd * DKV:(hd + 1) * DKV]
        vh = vb[:, hd * DKV:(hd + 1) * DKV]
        sc = jax.lax.dot_general(qh, kh, (((1,), (1,)), ((), ())),
                                 preferred_element_type=jnp.float32) / jnp.sqrt(
                                     jnp.float32(DKV))
        pr = jax.nn.softmax(sc, -1)
        outs.append(jnp.dot(pr, vh, preferred_element_type=jnp.float32))
    o_ref[0] = jnp.concatenate(outs, axis=-1)


def _oproj_body(h_ref, o_ref, wo_ref, bo_ref, s2_ref, b2_ref,
                ho_ref, hn2_ref):
    hh = h_ref[...] + (_dot(_bf(o_ref[...]), wo_ref[...]) + bo_ref[...])
    ho_ref[...] = hh
    hn2_ref[...] = _ln(hh, s2_ref[...], b2_ref[...])


def _route1_body(tok_ref, gw_ref, probs_ref, m1_ref, m2_ref):
    tok = tok_ref[...]
    logits = _dot(_bf(tok), gw_ref[...])
    probs = jax.nn.softmax(logits, -1)
    rows = jax.lax.broadcasted_iota(jnp.int32, (TP, E), 0)
    cols = jax.lax.broadcasted_iota(jnp.int32, (TP, E), 1)
    live = (rows < T).astype(jnp.float32)
    mx1 = jnp.max(probs, axis=-1, keepdims=True)
    e1 = jnp.min(jnp.where(probs == mx1, cols, E), axis=-1, keepdims=True)
    m1 = (cols == e1).astype(jnp.float32) * live
    p2 = probs * (1.0 - m1)
    mx2 = jnp.max(p2, axis=-1, keepdims=True)
    e2 = jnp.min(jnp.where(p2 == mx2, cols, E), axis=-1, keepdims=True)
    m2 = (cols == e2).astype(jnp.float32) * live
    probs_ref[...] = probs
    m1_ref[...] = m1
    m2_ref[...] = m2


def _route2_body(probs_ref, m1_ref, m2_ref, c1_ref, c2_ref, g1_ref, g2_ref):
    pid = pl.program_id(0)
    base = pid * TBLK
    m1 = m1_ref[...]
    m2 = m2_ref[...]
    rowi = jax.lax.broadcasted_iota(jnp.int32, (TBLK, TP), 0) + base
    colj = jax.lax.broadcasted_iota(jnp.int32, (TBLK, TP), 1)
    L = (colj < rowi).astype(jnp.float32)
    loc1 = jnp.dot(L, m1, preferred_element_type=jnp.float32)
    loc2 = (jnp.dot(L, m2, preferred_element_type=jnp.float32)
            + jnp.sum(m1, axis=0, keepdims=True))
    m1b = m1_ref[pl.ds(base, TBLK), :]
    m2b = m2_ref[pl.ds(base, TBLK), :]
    pb = probs_ref[pl.ds(base, TBLK), :]
    ecols = jax.lax.broadcasted_iota(jnp.int32, (TBLK, E), 1).astype(jnp.float32)
    l1 = jnp.sum(loc1 * m1b, axis=-1)
    l2 = jnp.sum(loc2 * m2b, axis=-1)
    v1 = l1 < float(CAP)
    v2 = l2 < float(CAP)
    g1r = jnp.where(v1, jnp.sum(pb * m1b, axis=-1), 0.0)
    g2r = jnp.where(v2, jnp.sum(pb * m2b, axis=-1), 0.0)
    den = g1r + g2r + 1e-9
    g1 = g1r / den
    g2 = g2r / den
    e1 = jnp.sum(ecols * m1b, axis=-1).astype(jnp.int32)
    e2 = jnp.sum(ecols * m2b, axis=-1).astype(jnp.int32)
    k1 = v1 & (g1 > 0.0)
    k2 = v2 & (g2 > 0.0)
    c1 = jnp.where(k1, e1 * CAPP + l1.astype(jnp.int32), -1)
    c2 = jnp.where(k2, e2 * CAPP + l2.astype(jnp.int32), -1)
    c1_ref[0, 0, :] = c1
    c2_ref[0, 0, :] = c2
    g1_ref[0, 0, :] = g1
    g2_ref[0, 0, :] = g2


def _dispatch_body(c1_ref, c2_ref, tok_ref, o_ref):
    pid = pl.program_id(0)
    rowid = jax.lax.broadcasted_iota(jnp.int32, (SBLK, TP), 0) + pid * SBLK
    c1 = c1_ref[...][None, :]
    c2 = c2_ref[...][None, :]
    ind = ((c1 == rowid) | (c2 == rowid)).astype(jnp.float32)
    o_ref[...] = jnp.dot(ind, _bf(tok_ref[...]),
                         preferred_element_type=jnp.float32)


def _ffn_body(x_ref, w1_ref, b1_ref, w2_ref, b2_ref, o_ref):
    x = x_ref[0]
    h = jax.nn.gelu(_dot(x, w1_ref[0]) + b1_ref[0])
    o_ref[0] = _dot(h, w2_ref[0]) + b2_ref[0]


def _combine_body(c1_ref, c2_ref, g1_ref, g2_ref, eo_ref, h_ref, o_ref):
    c1 = c1_ref[0, 0, :][:, None]
    c2 = c2_ref[0, 0, :][:, None]
    g1 = g1_ref[0, 0, :][:, None]
    g2 = g2_ref[0, 0, :][:, None]
    slot = jax.lax.broadcasted_iota(jnp.int32, (TBLK, NSLOT), 1)
    M = (jnp.where(slot == c1, g1, 0.0) + jnp.where(slot == c2, g2, 0.0))
    o_ref[...] = h_ref[...] + jnp.dot(M, _bf(eo_ref[...]),
                                      preferred_element_type=jnp.float32)


def _head_body(h_ref, s_ref, b_ref, wc_ref, bc_ref, o_ref):
    hn = _ln(h_ref[...], s_ref[...], b_ref[...])
    brow = jax.lax.broadcasted_iota(jnp.int32, (B, T), 0)
    scol = jax.lax.broadcasted_iota(jnp.int32, (B, T), 1)
    sel = ((scol >= brow * S) & (scol < (brow + 1) * S)).astype(jnp.float32) / S
    pooled = jnp.dot(sel, hn, preferred_element_type=jnp.float32,
                     precision=jax.lax.Precision.HIGHEST)
    o_ref[...] = jnp.dot(pooled, wc_ref[...],
                         preferred_element_type=jnp.float32) + bc_ref[...]


def kernel(x, Wpatch, bpatch, cls_tok, pos_emb, ln1_s, ln1_b, ln2_s, ln2_b,
           Wq, bq, Wk, bk, Wv, bv, Wo, bo, gate_w, W1, b1, W2, b2,
           lnf_s, lnf_b, Wc, bc):
    f32 = jnp.float32
    n = x.shape[0]
    g = IMG // P
    patches = x.reshape(n, 3, g, P, g, P).transpose(0, 2, 4, 1, 3, 5)
    patches = patches.reshape(n * g * g, 3 * P * P)

    emb = _pc(_embed_body, None, None, None,
              jax.ShapeDtypeStruct((n * g * g, HID), f32))(
        patches, Wpatch, bpatch)
    emb = emb.reshape(n, g * g, HID)
    h = jnp.concatenate([jnp.broadcast_to(cls_tok, (n, 1, HID)), emb], 1)
    h = h + pos_emb

    wspec = lambda shp: pl.BlockSpec(shp, lambda b: (0,) * len(shp))
    D3 = HEADS * DKV

    hf = h.reshape(T, HID)
    for i in range(DEPTH):
        q, k, v = _pc(
            _qkv_body, None, None, None,
            [jax.ShapeDtypeStruct((T, D3), f32)] * 3)(
            hf, ln1_s[i], ln1_b[i], Wq, bq, Wk, bk, Wv, bv)

        o = _pc(
            _mix_body, (B,),
            [pl.BlockSpec((1, S, D3), lambda b: (b, 0, 0))] * 3,
            pl.BlockSpec((1, S, D3), lambda b: (b, 0, 0)),
            jax.ShapeDtypeStruct((B, S, D3), f32))(
            q.reshape(B, S, D3), k.reshape(B, S, D3), v.reshape(B, S, D3))

        hf, hn2 = _pc(
            _oproj_body, None, None, None,
            [jax.ShapeDtypeStruct((T, HID), f32),
             jax.ShapeDtypeStruct((T, HID), f32)])(
            hf, o.reshape(T, D3), Wo, bo, ln2_s[i], ln2_b[i])

        tok = jnp.pad(hn2, ((0, TP - T), (0, 0)))
        probs, m1, m2 = _pc(
            _route1_body, None, None, None,
            [jax.ShapeDtypeStruct((TP, E), f32)] * 3)(tok, gate_w)

        c1, c2, g1, g2 = _pc(
            _route2_body, (TP // TBLK,),
            [wspec((TP, E)), wspec((TP, E)), wspec((TP, E))],
            [pl.BlockSpec((1, 1, TBLK), lambda b: (b, 0, 0))] * 4,
            [jax.ShapeDtypeStruct((TP // TBLK, 1, TBLK), jnp.int32),
             jax.ShapeDtypeStruct((TP // TBLK, 1, TBLK), jnp.int32),
             jax.ShapeDtypeStruct((TP // TBLK, 1, TBLK), f32),
             jax.ShapeDtypeStruct((TP // TBLK, 1, TBLK), f32)])(
            probs, m1, m2)

        c1f = c1.reshape(TP)
        c2f = c2.reshape(TP)
        ein = _pc(
            _dispatch_body, (NSLOT // SBLK,),
            [wspec((TP,)), wspec((TP,)), wspec((TP, HID))],
            pl.BlockSpec((SBLK, HID), lambda b: (b, 0)),
            jax.ShapeDtypeStruct((NSLOT, HID), f32))(c1f, c2f, tok)

        eo = _pc(
            _ffn_body, (E,),
            [pl.BlockSpec((1, CAPP, HID), lambda e: (e, 0, 0)),
             pl.BlockSpec((1, HID, DFF), lambda e: (e, 0, 0)),
             pl.BlockSpec((1, 1, DFF), lambda e: (e, 0, 0)),
             pl.BlockSpec((1, DFF, HID), lambda e: (e, 0, 0)),
             pl.BlockSpec((1, 1, HID), lambda e: (e, 0, 0))],
            pl.BlockSpec((1, CAPP, HID), lambda e: (e, 0, 0)),
            jax.ShapeDtypeStruct((E, CAPP, HID), f32))(
            ein.reshape(E, CAPP, HID), W1, b1.reshape(E, 1, DFF),
            W2, b2.reshape(E, 1, HID))

        hpad = jnp.pad(hf, ((0, TP - T), (0, 0)))
        hnext = _pc(
            _combine_body, (TP // TBLK,),
            [pl.BlockSpec((1, 1, TBLK), lambda b: (b, 0, 0))] * 4
            + [wspec((NSLOT, HID)),
               pl.BlockSpec((TBLK, HID), lambda b: (b, 0))],
            pl.BlockSpec((TBLK, HID), lambda b: (b, 0)),
            jax.ShapeDtypeStruct((TP, HID), f32))(
            c1, c2, g1, g2, eo.reshape(NSLOT, HID), hpad)
        hf = hnext[:T]

    logits = _pc(_head_body, None, None, None,
                 jax.ShapeDtypeStruct((B, NCLS), f32))(
        hf, lnf_s, lnf_b, Wc, bc)
    return logits
